# baseline (speedup 1.0000x reference)
import jax
import jax.numpy as jnp
from jax.experimental import pallas as pl


def _gcn2_kernel(x_ref, adj_ref, w1_ref, b1_ref, w2_ref, b2_ref, out_ref):
    adjv = adj_ref[...]
    n = adjv.shape[0]
    bt = x_ref.shape[0]
    f = x_ref.shape[2]
    dis = jax.lax.rsqrt(1.0 + jnp.sum(adjv, axis=0, keepdims=True))
    r = jax.lax.broadcasted_iota(jnp.int32, (n, n), 0)
    c = jax.lax.broadcasted_iota(jnp.int32, (n, n), 1)
    ahat = (adjv + jnp.where(r == c, 1.0, 0.0)) * dis

    w1t = w1_ref[...].T
    w2t = w2_ref[...].T
    b1 = b1_ref[...]
    b2 = b2_ref[...]

    xp = jnp.concatenate([x_ref[b, :, :].T for b in range(bt)], axis=0)

    agg1 = jnp.dot(xp * dis, ahat, preferred_element_type=jnp.float32)
    h1 = jnp.concatenate(
        [
            jnp.maximum(
                jnp.dot(w1t, agg1[b * f:(b + 1) * f, :],
                        preferred_element_type=jnp.float32) + b1,
                0.0,
            )
            for b in range(bt)
        ],
        axis=0,
    )

    h = w1t.shape[0]
    agg2 = jnp.dot(h1 * dis, ahat, preferred_element_type=jnp.float32)
    for b in range(bt):
        piece = jnp.maximum(
            jnp.dot(w2t, agg2[b * h:(b + 1) * h, :],
                    preferred_element_type=jnp.float32) + b2,
            0.0,
        )
        out_ref[b, :, :] = piece.T


def kernel(x, adj, W1, b1, W2, b2):
    bt, n, _ = x.shape
    o = W2.shape[1]
    return pl.pallas_call(
        _gcn2_kernel,
        out_shape=jax.ShapeDtypeStruct((bt, n, o), jnp.float32),
    )(x, adj, W1, b1[:, None], W2, b2[:, None])


# speedup vs baseline: 1.6959x; 1.6959x over previous
import jax
import jax.numpy as jnp
from jax.experimental import pallas as pl


def _gcn2_kernel(xp_ref, adj_ref, w1t_ref, b1_ref, w2t_ref, b2_ref, out_ref):
    adjv = adj_ref[...]
    n = adjv.shape[0]
    dis = jax.lax.rsqrt(1.0 + jnp.sum(adjv, axis=0, keepdims=True))
    r = jax.lax.broadcasted_iota(jnp.int32, (n, n), 0)
    c = jax.lax.broadcasted_iota(jnp.int32, (n, n), 1)
    ahat = (adjv + jnp.where(r == c, 1.0, 0.0)) * dis

    w1t = w1t_ref[...]
    w2t = w2t_ref[...]
    f1 = w1t.shape[1]
    f2 = w2t.shape[1]
    nb = xp_ref.shape[0] // f1

    def layer(hcur, wt, f, bias):
        agg = jnp.dot(hcur * dis, ahat, preferred_element_type=jnp.float32)
        pieces = [
            jnp.dot(wt, agg[i * f:(i + 1) * f, :],
                    preferred_element_type=jnp.float32)
            for i in range(nb)
        ]
        z = jnp.concatenate(pieces, axis=0)
        return jnp.maximum(z + bias, 0.0)

    h1 = layer(xp_ref[...], w1t, f1, b1_ref[...])
    out_ref[...] = layer(h1, w2t, f2, b2_ref[...])


def kernel(x, adj, W1, b1, W2, b2):
    bt, n, f = x.shape
    o = W2.shape[1]
    xp = x.transpose(0, 2, 1).reshape(bt * f, n)
    b1c = jnp.tile(b1, bt)[:, None]
    b2c = jnp.tile(b2, bt)[:, None]
    outp = pl.pallas_call(
        _gcn2_kernel,
        out_shape=jax.ShapeDtypeStruct((bt * o, n), jnp.float32),
    )(xp, adj, W1.T, b1c, W2.T, b2c)
    return outp.reshape(bt, o, n).transpose(0, 2, 1)
